# baseline (speedup 1.0000x reference)
import functools

import jax
import jax.numpy as jnp
from jax import lax
from jax.experimental import pallas as pl
from jax.experimental.pallas import tpu as pltpu
from jax.experimental.pallas import tpu_sc as plsc

_ROWS = 512
_NW = 32
_WIN = 256
_RCHUNK = 64


def _sigmoid16(x):
    return 1.0 / (1.0 + jnp.exp(-x))


def _sc_body(rows, cols, logits_hbm, idx_hbm, tk_hbm, upd_hbm,
             idx_v, tk_v, buf, out_v):
    rpw = rows // _NW
    wid = lax.axis_index("s") * 2 + lax.axis_index("c")
    row0 = wid * rpw

    pltpu.sync_copy(idx_hbm, idx_v)
    pltpu.sync_copy(tk_hbm, tk_v)

    for chunk in range(rpw // _RCHUNK):
        r0 = row0 + chunk * _RCHUNK
        pltpu.sync_copy(logits_hbm.at[pl.ds(r0, _RCHUNK), :], buf)

        def row_step(r, _):
            ro = chunk * _RCHUNK + r
            rv = jnp.full((16,), r, jnp.int32)

            def grab(g, h):
                ci = idx_v[pl.ds(32 * g + 16 * h, 16)]
                return _sigmoid16(plsc.load_gather(buf, [rv, ci]))

            for h in (0, 1):
                tau = tk_v[pl.ds(16 * h, 16)]
                kap = tk_v[pl.ds(32 + 16 * h, 16)]
                qi = grab(0, h)
                qj = grab(1, h)
                out_v[ro, pl.ds(16 * h, 16)] = jnp.minimum(
                    jnp.maximum(qj, qi + tau), 1.0
                )
                qi = grab(2, h)
                qj = grab(3, h)
                s = qi + qj
                dd = qi - qj
                viol = s > kap
                cap = jnp.minimum(kap, 1.0)
                qin = jnp.where(
                    viol, jnp.clip((kap + dd) * 0.5, 0.0, cap), qi
                )
                qjn = jnp.where(
                    viol, jnp.clip((kap - dd) * 0.5, 0.0, cap), qj
                )
                out_v[ro, pl.ds(32 + 16 * h, 16)] = qin
                out_v[ro, pl.ds(64 + 16 * h, 16)] = qjn
            return 0

        lax.fori_loop(0, _RCHUNK, row_step, 0)

    pltpu.sync_copy(out_v, upd_hbm.at[pl.ds(row0, rpw), :])


def _sc_project(logits, idx_all, tk):
    rows, cols = logits.shape
    rpw = rows // _NW
    mesh = plsc.VectorSubcoreMesh(core_axis_name="c", subcore_axis_name="s")
    return pl.kernel(
        functools.partial(_sc_body, rows, cols),
        out_type=jax.ShapeDtypeStruct((rows, 96), jnp.float32),
        mesh=mesh,
        compiler_params=pltpu.CompilerParams(
            use_tc_tiling_on_sc=False, needs_layout_passes=False
        ),
        scratch_types=[
            pltpu.VMEM((128,), jnp.int32),
            pltpu.VMEM((64,), jnp.float32),
            pltpu.VMEM((_RCHUNK, cols), jnp.float32),
            pltpu.VMEM((rpw, 96), jnp.float32),
        ],
    )(logits, idx_all, tk)


def _merge_body(sidx_ref, upd_ref, x_ref, o_ref):
    x = x_ref[...]
    p = jax.nn.sigmoid(x)
    cols = x.shape[1]
    scol = sidx_ref[:, 0:1]
    ssel = (lax.broadcasted_iota(jnp.int32, (96, cols), 1) == scol).astype(
        jnp.float32
    )
    mask = jnp.sum(ssel, axis=0, keepdims=True)
    scat = lax.dot_general(
        upd_ref[...], ssel, (((1,), (0,)), ((), ())),
        preferred_element_type=jnp.float32,
    )
    o_ref[...] = p * (1.0 - mask) + scat


def kernel(logits, imp_tau, exc_kappa, imp_i, imp_j, exc_i, exc_j):
    rows, cols = logits.shape
    idx_all = jnp.concatenate([imp_i, imp_j, exc_i, exc_j]).astype(jnp.int32)
    tk = jnp.concatenate(
        [imp_tau.astype(jnp.float32), exc_kappa.astype(jnp.float32)]
    )

    upd = _sc_project(logits, idx_all, tk)

    sidx2 = jnp.broadcast_to(idx_all[32:, None], (96, 128))
    grid = rows // _ROWS
    return pl.pallas_call(
        _merge_body,
        grid=(grid,),
        in_specs=[
            pl.BlockSpec((96, 128), lambda i: (0, 0)),
            pl.BlockSpec((_ROWS, 96), lambda i: (i, 0)),
            pl.BlockSpec((_ROWS, cols), lambda i: (i, 0)),
        ],
        out_specs=pl.BlockSpec((_ROWS, cols), lambda i: (i, 0)),
        out_shape=jax.ShapeDtypeStruct((rows, cols), jnp.float32),
    )(sidx2, upd, logits)


# speedup vs baseline: 2.5797x; 2.5797x over previous
import functools

import jax
import jax.numpy as jnp
from jax import lax
from jax.experimental import pallas as pl
from jax.experimental.pallas import tpu as pltpu
from jax.experimental.pallas import tpu_sc as plsc

_BATCH_BLK = 512
_NW = 32
_WINR = 40
_LANES = 128


def _sig_body(x_ref, o_ref):
    o_ref[...] = jax.nn.sigmoid(x_ref[...])


def _sc_body(probs_ref, idx_hbm, taub_hbm, kapb_hbm,
             idx_v, taub_v, kapb_v, b0, b1, b2, b3):
    wid = lax.axis_index("s") * 2 + lax.axis_index("c")
    l0 = pl.multiple_of(wid * _LANES, _LANES)

    pltpu.sync_copy(idx_hbm, idx_v)
    pltpu.sync_copy(taub_hbm, taub_v)
    pltpu.sync_copy(kapb_hbm, kapb_v)

    starts, offs = [], []
    for g in range(4):
        r0 = jnp.min(idx_v[pl.ds(32 * g, 16)])
        r0a = jnp.bitwise_and(r0, -8)
        starts.append(pl.multiple_of(r0a, 8))
        offs.append(r0 - r0a)

    bufs = (b0, b1, b2, b3)
    for g in range(4):
        pltpu.sync_copy(
            probs_ref.at[pl.ds(starts[g], _WINR), pl.ds(l0, _LANES)], bufs[g]
        )

    def pair_step(k, _):
        tau = taub_v[k]
        kap = kapb_v[k]
        for l in range(_LANES // 16):
            sl = pl.ds(16 * l, 16)
            qi = b0[offs[0] + k, sl]
            qj = b1[offs[1] + k, sl]
            b1[offs[1] + k, sl] = jnp.minimum(
                jnp.maximum(qj, qi + tau), 1.0
            )
            qi = b2[offs[2] + k, sl]
            qj = b3[offs[3] + k, sl]
            s = qi + qj
            dd = qi - qj
            viol = s > kap
            cap = jnp.minimum(kap, 1.0)
            b2[offs[2] + k, sl] = jnp.where(
                viol, jnp.clip((kap + dd) * 0.5, 0.0, cap), qi
            )
            b3[offs[3] + k, sl] = jnp.where(
                viol, jnp.clip((kap - dd) * 0.5, 0.0, cap), qj
            )
        return 0

    lax.fori_loop(0, 32, pair_step, 0)

    for g in range(1, 4):
        pltpu.sync_copy(
            bufs[g], probs_ref.at[pl.ds(starts[g], _WINR), pl.ds(l0, _LANES)]
        )


def _sc_update(probs_ref, idx_all, taub, kapb):
    mesh = plsc.VectorSubcoreMesh(core_axis_name="c", subcore_axis_name="s")
    pl.kernel(
        _sc_body,
        out_type=(),
        mesh=mesh,
        compiler_params=pltpu.CompilerParams(
            use_tc_tiling_on_sc=True, needs_layout_passes=False
        ),
        scratch_types=[
            pltpu.VMEM((128,), jnp.int32),
            pltpu.VMEM((32, 16), jnp.float32),
            pltpu.VMEM((32, 16), jnp.float32),
            pltpu.VMEM((_WINR, _LANES), jnp.float32),
            pltpu.VMEM((_WINR, _LANES), jnp.float32),
            pltpu.VMEM((_WINR, _LANES), jnp.float32),
            pltpu.VMEM((_WINR, _LANES), jnp.float32),
        ],
    )(probs_ref, idx_all, taub, kapb)


def kernel(logits, imp_tau, exc_kappa, imp_i, imp_j, exc_i, exc_j):
    lt = logits.T
    cols, batch = lt.shape

    probs_t = pl.pallas_call(
        _sig_body,
        grid=(batch // _BATCH_BLK,),
        in_specs=[pl.BlockSpec((cols, _BATCH_BLK), lambda i: (0, i))],
        out_specs=pl.BlockSpec((cols, _BATCH_BLK), lambda i: (0, i)),
        out_shape=jax.ShapeDtypeStruct((cols, batch), jnp.float32),
    )(lt)

    idx_all = jnp.concatenate([imp_i, imp_j, exc_i, exc_j]).astype(jnp.int32)
    taub = jnp.broadcast_to(
        imp_tau.astype(jnp.float32)[:, None], (32, 16)
    )
    kapb = jnp.broadcast_to(
        exc_kappa.astype(jnp.float32)[:, None], (32, 16)
    )

    r = jax.new_ref(probs_t)
    _sc_update(r, idx_all, taub, kapb)
    return r[...].T


# speedup vs baseline: 2.7851x; 1.0796x over previous
import functools

import jax
import jax.numpy as jnp
from jax import lax
from jax.experimental import pallas as pl
from jax.experimental.pallas import tpu as pltpu
from jax.experimental.pallas import tpu_sc as plsc

_BATCH_BLK = 512
_NW = 32
_WINR = 40
_LANES = 128


def _sig_body(x_ref, o_ref):
    o_ref[...] = jax.nn.sigmoid(x_ref[...])


def _sc_body(probs_ref, idx_hbm, tk_hbm, idx_v, tk_v, b0, b1, b2, b3, sem):
    wid = lax.axis_index("s") * 2 + lax.axis_index("c")
    l0 = pl.multiple_of(wid * _LANES, _LANES)

    cp1 = pltpu.async_copy(idx_hbm, idx_v, sem)
    cp2 = pltpu.async_copy(tk_hbm, tk_v, sem)
    cp1.wait()
    cp2.wait()

    starts, offs = [], []
    for g in range(4):
        r0 = jnp.min(idx_v[pl.ds(32 * g, 16)])
        r0a = jnp.bitwise_and(r0, -8)
        starts.append(pl.multiple_of(r0a, 8))
        offs.append(r0 - r0a)

    bufs = (b0, b1, b2, b3)
    cps = [
        pltpu.async_copy(
            probs_ref.at[pl.ds(starts[g], _WINR), pl.ds(l0, _LANES)],
            bufs[g],
            sem,
        )
        for g in range(4)
    ]
    for cp in cps:
        cp.wait()

    def pair_step(k, _):
        tau = tk_v[k]
        kap = tk_v[32 + k]
        for l in range(_LANES // 16):
            sl = pl.ds(16 * l, 16)
            qi = b0[offs[0] + k, sl]
            qj = b1[offs[1] + k, sl]
            b1[offs[1] + k, sl] = jnp.minimum(
                jnp.maximum(qj, qi + tau), 1.0
            )
            qi = b2[offs[2] + k, sl]
            qj = b3[offs[3] + k, sl]
            s = qi + qj
            dd = qi - qj
            viol = s > kap
            cap = jnp.minimum(kap, 1.0)
            b2[offs[2] + k, sl] = jnp.where(
                viol, jnp.clip((kap + dd) * 0.5, 0.0, cap), qi
            )
            b3[offs[3] + k, sl] = jnp.where(
                viol, jnp.clip((kap - dd) * 0.5, 0.0, cap), qj
            )
        return 0

    lax.fori_loop(0, 32, pair_step, 0, unroll=4)

    cps = [
        pltpu.async_copy(
            bufs[g],
            probs_ref.at[pl.ds(starts[g], _WINR), pl.ds(l0, _LANES)],
            sem,
        )
        for g in range(1, 4)
    ]
    for cp in cps:
        cp.wait()


def _sc_update(probs_ref, idx_all, tkb):
    mesh = plsc.VectorSubcoreMesh(core_axis_name="c", subcore_axis_name="s")
    pl.kernel(
        _sc_body,
        out_type=(),
        mesh=mesh,
        compiler_params=pltpu.CompilerParams(
            use_tc_tiling_on_sc=True, needs_layout_passes=False
        ),
        scratch_types=[
            pltpu.VMEM((128,), jnp.int32),
            pltpu.VMEM((64, 16), jnp.float32),
            pltpu.VMEM((_WINR, _LANES), jnp.float32),
            pltpu.VMEM((_WINR, _LANES), jnp.float32),
            pltpu.VMEM((_WINR, _LANES), jnp.float32),
            pltpu.VMEM((_WINR, _LANES), jnp.float32),
            pltpu.SemaphoreType.DMA,
        ],
    )(probs_ref, idx_all, tkb)


def kernel(logits, imp_tau, exc_kappa, imp_i, imp_j, exc_i, exc_j):
    lt = logits.T
    cols, batch = lt.shape

    probs_t = pl.pallas_call(
        _sig_body,
        grid=(batch // _BATCH_BLK,),
        in_specs=[pl.BlockSpec((cols, _BATCH_BLK), lambda i: (0, i))],
        out_specs=pl.BlockSpec((cols, _BATCH_BLK), lambda i: (0, i)),
        out_shape=jax.ShapeDtypeStruct((cols, batch), jnp.float32),
    )(lt)

    idx_all = jnp.concatenate([imp_i, imp_j, exc_i, exc_j]).astype(jnp.int32)
    tkb = jnp.broadcast_to(
        jnp.concatenate(
            [imp_tau.astype(jnp.float32), exc_kappa.astype(jnp.float32)]
        )[:, None],
        (64, 16),
    )

    r = jax.new_ref(probs_t)
    _sc_update(r, idx_all, tkb)
    return r[...].T


# speedup vs baseline: 2.8578x; 1.0261x over previous
import jax
import jax.numpy as jnp
from jax import lax
from jax.experimental import pallas as pl
from jax.experimental.pallas import tpu as pltpu
from jax.experimental.pallas import tpu_sc as plsc

_BATCH_BLK = 1024
_WINR = 40
_LANES = 128


def _sig_body(x_ref, o_ref):
    o_ref[...] = jax.nn.sigmoid(x_ref[...])


def _sc_body(probs_ref, idx_hbm, tk_hbm, idx_v, tk_v, b0, b1, b2, b3, sem):
    wid = lax.axis_index("s") * 2 + lax.axis_index("c")
    l0 = pl.multiple_of(wid * _LANES, _LANES)

    cp1 = pltpu.async_copy(idx_hbm, idx_v, sem)
    cp2 = pltpu.async_copy(tk_hbm, tk_v, sem)
    cp1.wait()
    cp2.wait()

    starts, offs = [], []
    for g in range(4):
        r0 = jnp.min(idx_v[pl.ds(32 * g, 16)])
        r0a = jnp.bitwise_and(r0, -8)
        starts.append(pl.multiple_of(r0a, 8))
        offs.append(r0 - r0a)

    bufs = (b0, b1, b2, b3)
    cps = [
        pltpu.async_copy(
            probs_ref.at[pl.ds(starts[g], _WINR), pl.ds(l0, _LANES)],
            bufs[g],
            sem,
        )
        for g in range(4)
    ]
    for cp in cps:
        cp.wait()

    def pair_step(k, _):
        tau = tk_v[k]
        kap = tk_v[32 + k]
        for l in range(_LANES // 16):
            sl = pl.ds(16 * l, 16)
            qi = b0[offs[0] + k, sl]
            qj = b1[offs[1] + k, sl]
            b1[offs[1] + k, sl] = jnp.minimum(
                jnp.maximum(qj, qi + tau), 1.0
            )
            qi = b2[offs[2] + k, sl]
            qj = b3[offs[3] + k, sl]
            s = qi + qj
            dd = qi - qj
            viol = s > kap
            cap = jnp.minimum(kap, 1.0)
            b2[offs[2] + k, sl] = jnp.where(
                viol, jnp.clip((kap + dd) * 0.5, 0.0, cap), qi
            )
            b3[offs[3] + k, sl] = jnp.where(
                viol, jnp.clip((kap - dd) * 0.5, 0.0, cap), qj
            )
        return 0

    lax.fori_loop(0, 32, pair_step, 0, unroll=4)

    cps = [
        pltpu.async_copy(
            bufs[g],
            probs_ref.at[pl.ds(starts[g], _WINR), pl.ds(l0, _LANES)],
            sem,
        )
        for g in range(1, 4)
    ]
    for cp in cps:
        cp.wait()


def _sc_update(probs_ref, idx_all, tkb):
    mesh = plsc.VectorSubcoreMesh(core_axis_name="c", subcore_axis_name="s")
    pl.kernel(
        _sc_body,
        out_type=(),
        mesh=mesh,
        compiler_params=pltpu.CompilerParams(
            use_tc_tiling_on_sc=True, needs_layout_passes=False
        ),
        scratch_types=[
            pltpu.VMEM((128,), jnp.int32),
            pltpu.VMEM((64, 16), jnp.float32),
            pltpu.VMEM((_WINR, _LANES), jnp.float32),
            pltpu.VMEM((_WINR, _LANES), jnp.float32),
            pltpu.VMEM((_WINR, _LANES), jnp.float32),
            pltpu.VMEM((_WINR, _LANES), jnp.float32),
            pltpu.SemaphoreType.DMA,
        ],
    )(probs_ref, idx_all, tkb)


def kernel(logits, imp_tau, exc_kappa, imp_i, imp_j, exc_i, exc_j):
    lt = logits.T
    cols, batch = lt.shape

    probs_t = pl.pallas_call(
        _sig_body,
        grid=(batch // _BATCH_BLK,),
        in_specs=[pl.BlockSpec((cols, _BATCH_BLK), lambda i: (0, i))],
        out_specs=pl.BlockSpec((cols, _BATCH_BLK), lambda i: (0, i)),
        out_shape=jax.ShapeDtypeStruct((cols, batch), jnp.float32),
    )(lt)

    idx_all = jnp.concatenate([imp_i, imp_j, exc_i, exc_j]).astype(jnp.int32)
    tkb = jnp.broadcast_to(
        jnp.concatenate(
            [imp_tau.astype(jnp.float32), exc_kappa.astype(jnp.float32)]
        )[:, None],
        (64, 16),
    )

    r = jax.new_ref(probs_t)
    _sc_update(r, idx_all, tkb)
    return r[...].T
